# baseline (speedup 1.0000x reference)
import functools

import jax
import jax.numpy as jnp
from jax import lax
from jax.experimental import pallas as pl
from jax.experimental.pallas import tpu as pltpu
from jax.experimental.pallas import tpu_sc as plsc

_OMEGA = 64
_SINK = 4
_MAX_WINDOWS = 30000

_NC = 2
_NS = 16
_ROW = _MAX_WINDOWS * 3
_CHUNK = 16 * _ROW // (_NC * _NS)
_PREF = 96
_FILL = 7200
_TAIL = _CHUNK - _PREF - 6 * _FILL


def _reduce_body(nq, nw, score_end, attn_ref, out_ref, acc_ref):
    q = pl.program_id(1)

    @pl.when(q == 0)
    def _init():
        acc_ref[...] = jnp.zeros_like(acc_ref)

    acc_ref[...] += jnp.sum(attn_ref[0], axis=0, keepdims=True)

    @pl.when(q == nq - 1)
    def _finish():
        s = acc_ref.shape[1]
        npat = 3 * nw
        c = lax.broadcasted_iota(jnp.int32, (s, 128), 0)
        i = lax.broadcasted_iota(jnp.int32, (s, 128), 1)
        live = (i % 3 == 0) & (i < npat)
        col_in = (c >= _SINK) & (c < score_end) & ((c - _SINK) // _OMEGA == i // 3)
        m = (live & col_in).astype(jnp.float32)
        winpart = lax.dot_general(
            acc_ref[...], m, (((1,), (0,)), ((), ())),
            preferred_element_type=jnp.float32)
        i2 = lax.broadcasted_iota(jnp.int32, (1, 128), 1)
        idxf = (i2 // 3).astype(jnp.float32)
        res = jnp.where(i2 % 3 == 0, winpart, idxf)
        row = pl.program_id(0)
        out_ref[pl.ds(row, 1), :] = jnp.where(i2 < npat, res, jnp.float32(jnp.nan))


def _reduce(attn):
    h, s, _ = attn.shape
    qc = 256
    nq = s // qc
    nw = (s - _SINK) // _OMEGA
    score_end = _SINK + nw * _OMEGA
    return pl.pallas_call(
        functools.partial(_reduce_body, nq, nw, score_end),
        grid=(h, nq),
        in_specs=[pl.BlockSpec((1, qc, s), lambda hh, qq: (hh, qq, 0))],
        out_specs=pl.BlockSpec((h, 128), lambda hh, qq: (0, 0)),
        out_shape=jax.ShapeDtypeStruct((h, 128), jnp.float32),
        scratch_shapes=[pltpu.VMEM((1, s), jnp.float32)],
        compiler_params=pltpu.CompilerParams(
            dimension_semantics=("parallel", "arbitrary")),
    )(attn)


def _assemble_body(prefix_hbm, out_hbm, fillbuf, pbuf):
    cid = lax.axis_index("c")
    sid = lax.axis_index("s")
    wid = sid * _NC + cid
    base = pl.multiple_of(wid * _CHUNK, 8)
    head = wid // 2

    nanv = jnp.full((16,), jnp.nan, dtype=jnp.float32)

    def fill(i, carry):
        fillbuf[pl.ds(i * 16, 16)] = nanv
        return carry

    lax.fori_loop(0, _FILL // 16, fill, 0)

    @pl.when(wid % 2 == 0)
    def _prefix():
        src = pl.multiple_of(head * 128, 8)
        pltpu.sync_copy(prefix_hbm.at[pl.ds(src, _PREF)], pbuf)
        pltpu.sync_copy(pbuf, out_hbm.at[pl.ds(base, _PREF)])

    @pl.when(wid % 2 == 1)
    def _nan_head():
        pltpu.sync_copy(fillbuf.at[pl.ds(0, _PREF)], out_hbm.at[pl.ds(base, _PREF)])

    def nan_stream(j, carry):
        off = pl.multiple_of(base + _PREF + j * _FILL, 8)
        pltpu.sync_copy(fillbuf, out_hbm.at[pl.ds(off, _FILL)])
        return carry

    lax.fori_loop(0, 6, nan_stream, 0)
    off = pl.multiple_of(base + _PREF + 6 * _FILL, 8)
    pltpu.sync_copy(fillbuf.at[pl.ds(0, _TAIL)], out_hbm.at[pl.ds(off, _TAIL)])


def _assemble(prefix_flat):
    mesh = plsc.VectorSubcoreMesh(
        core_axis_name="c", subcore_axis_name="s",
        num_cores=_NC, num_subcores=_NS)
    run = pl.kernel(
        _assemble_body,
        out_type=jax.ShapeDtypeStruct((16 * _ROW,), jnp.float32),
        mesh=mesh,
        scratch_types=[
            pltpu.VMEM((_FILL,), jnp.float32),
            pltpu.VMEM((_PREF,), jnp.float32),
        ],
    )
    return run(prefix_flat)


def kernel(past_key, past_value, attn_score_cache):
    b, h, s, _ = attn_score_cache.shape
    attn = attn_score_cache.reshape(h, s, s)
    prefix = _reduce(attn)
    flat = _assemble(prefix.reshape(-1))
    return flat.reshape(h, _MAX_WINDOWS, 3)


# speedup vs baseline: 3.7525x; 3.7525x over previous
import functools

import jax
import jax.numpy as jnp
from jax import lax
from jax.experimental import pallas as pl
from jax.experimental.pallas import tpu as pltpu

_OMEGA = 64
_SINK = 4
_MAX_WINDOWS = 30000


def _body(nh, nq, nw, score_end, attn_ref, out_ref, acc_ref, win_ref):
    h = pl.program_id(0)
    q = pl.program_id(1)
    s = acc_ref.shape[1]

    @pl.when(q == 0)
    def _init():
        acc_ref[...] = jnp.zeros_like(acc_ref)

    acc_ref[...] += jnp.sum(attn_ref[0], axis=0, keepdims=True)

    @pl.when(q == nq - 1)
    def _win():
        c = lax.broadcasted_iota(jnp.int32, (s, 128), 0)
        w = lax.broadcasted_iota(jnp.int32, (s, 128), 1)
        m = ((w < nw) & (c >= _SINK) & (c < score_end)
             & ((c - _SINK) // _OMEGA == w)).astype(jnp.float32)
        win_ref[pl.ds(h, 1), :] = lax.dot_general(
            acc_ref[...], m, (((1,), (0,)), ((), ())),
            preferred_element_type=jnp.float32)

    @pl.when((h == nh - 1) & (q == nq - 1))
    def _assemble():
        col = lax.broadcasted_iota(jnp.int32, (nh, _MAX_WINDOWS), 1)
        nanp = jnp.full((nh, _MAX_WINDOWS - 128), jnp.nan, dtype=jnp.float32)
        winpad = jnp.concatenate([win_ref[...], nanp], axis=1)
        live = col < nw
        out_ref[0] = jnp.where(live, winpad, jnp.float32(jnp.nan))
        idx_plane = jnp.where(live, col.astype(jnp.float32), jnp.float32(jnp.nan))
        out_ref[1] = idx_plane
        out_ref[2] = idx_plane


def _fused(attn):
    h, s, _ = attn.shape
    qc = 256
    nq = s // qc
    nw = (s - _SINK) // _OMEGA
    score_end = _SINK + nw * _OMEGA
    return pl.pallas_call(
        functools.partial(_body, h, nq, nw, score_end),
        grid=(h, nq),
        in_specs=[pl.BlockSpec((1, qc, s), lambda hh, qq: (hh, qq, 0))],
        out_specs=pl.BlockSpec((3, h, _MAX_WINDOWS), lambda hh, qq: (0, 0, 0)),
        out_shape=jax.ShapeDtypeStruct((3, h, _MAX_WINDOWS), jnp.float32),
        scratch_shapes=[
            pltpu.VMEM((1, s), jnp.float32),
            pltpu.VMEM((h, 128), jnp.float32),
        ],
        compiler_params=pltpu.CompilerParams(
            dimension_semantics=("arbitrary", "arbitrary")),
    )(attn)


def kernel(past_key, past_value, attn_score_cache):
    b, h, s, _ = attn_score_cache.shape
    attn = attn_score_cache.reshape(h, s, s)
    planes = _fused(attn)
    return jnp.transpose(planes, (1, 2, 0))
